# Initial kernel scaffold; baseline (speedup 1.0000x reference)
import jax
import jax.numpy as jnp
from jax.experimental import pallas as pl


def kernel(xyz, W1, b1, g1, be1, W2, b2, g2, be2):
    raise NotImplementedError("write your pallas kernel here")


# speedup vs baseline: 4.9861x; 4.9861x over previous
import functools

import jax
import jax.numpy as jnp
from jax.experimental import pallas as pl
from jax.experimental.pallas import tpu as pltpu

B = 8
N = 16384
S = 256
K = 32
C1 = 32
C2 = 64
BIG = 3e38


def _fps_kernel(xyz_ref, init_ref, nx_ref, ny_ref, nz_ref):
    x = xyz_ref[:, 0, :]
    y = xyz_ref[:, 1, :]
    z = xyz_ref[:, 2, :]
    iota_n = jax.lax.broadcasted_iota(jnp.int32, (B, N), 1)
    iota_s = jax.lax.broadcasted_iota(jnp.int32, (B, S), 1)

    def body(i, carry):
        distance, far, ax, ay, az = carry
        sel = iota_n == far
        cx = jnp.sum(jnp.where(sel, x, 0.0), axis=1, keepdims=True)
        cy = jnp.sum(jnp.where(sel, y, 0.0), axis=1, keepdims=True)
        cz = jnp.sum(jnp.where(sel, z, 0.0), axis=1, keepdims=True)
        rec = iota_s == i
        ax = ax + jnp.where(rec, cx, 0.0)
        ay = ay + jnp.where(rec, cy, 0.0)
        az = az + jnp.where(rec, cz, 0.0)
        dx = x - cx
        dy = y - cy
        dz = z - cz
        dist = (dx * dx + dy * dy) + dz * dz
        distance = jnp.where(dist < distance, dist, distance)
        m = jnp.max(distance, axis=1, keepdims=True)
        far = jnp.min(jnp.where(distance == m, iota_n, N), axis=1, keepdims=True)
        return distance, far, ax, ay, az

    dist0 = jnp.full((B, N), 1e10, dtype=jnp.float32)
    zs = jnp.zeros((B, S), dtype=jnp.float32)
    far0 = init_ref[...]
    _, _, ax, ay, az = jax.lax.fori_loop(0, S, body, (dist0, far0, zs, zs, zs))
    nx_ref[...] = ax
    ny_ref[...] = ay
    nz_ref[...] = az


def _run_fps(xyz, init_far):
    out = jax.ShapeDtypeStruct((B, S), jnp.float32)
    return pl.pallas_call(
        _fps_kernel,
        out_shape=(out, out, out),
    )(xyz, init_far)


def _select_kernel(xyz_ref, nx_ref, ny_ref, nz_ref, gx_ref, gy_ref, gz_ref,
                   d_ref):
    x = xyz_ref[:, 0, :]
    y = xyz_ref[:, 1, :]
    z = xyz_ref[:, 2, :]
    nxr = nx_ref[:, 0, :]
    nyr = ny_ref[:, 0, :]
    nzr = nz_ref[:, 0, :]

    io_l = jax.lax.broadcasted_iota(jnp.int32, (K, K), 1)
    io_s = jax.lax.broadcasted_iota(jnp.int32, (K, K), 0)
    t = io_l == io_s
    cx = jnp.sum(jnp.where(t, nxr[:, :K], 0.0), axis=1, keepdims=True)
    cy = jnp.sum(jnp.where(t, nyr[:, :K], 0.0), axis=1, keepdims=True)
    cz = jnp.sum(jnp.where(t, nzr[:, :K], 0.0), axis=1, keepdims=True)

    dx = x - cx
    dy = y - cy
    dz = z - cz
    d_ref[...] = (dx * dx + dy * dy) + dz * dz

    iota_n = jax.lax.broadcasted_iota(jnp.int32, (K, N), 1)
    iota_s = jax.lax.broadcasted_iota(jnp.int32, (K, S), 1)
    iota_s1 = jax.lax.broadcasted_iota(jnp.int32, (1, S), 1)

    def body(s, carry):
        gx, gy, gz = carry
        d = d_ref[...]
        m = jnp.min(d, axis=1, keepdims=True)
        idx = jnp.min(jnp.where(d == m, iota_n, N), axis=1, keepdims=True)
        fm = iota_n == idx
        d_ref[...] = jnp.where(fm, BIG, d)
        px = jnp.sum(jnp.where(fm, x, 0.0), axis=1, keepdims=True)
        py = jnp.sum(jnp.where(fm, y, 0.0), axis=1, keepdims=True)
        pz = jnp.sum(jnp.where(fm, z, 0.0), axis=1, keepdims=True)
        sm = iota_s1 == s
        nxs = jnp.sum(jnp.where(sm, nxr, 0.0), axis=1, keepdims=True)
        nys = jnp.sum(jnp.where(sm, nyr, 0.0), axis=1, keepdims=True)
        nzs = jnp.sum(jnp.where(sm, nzr, 0.0), axis=1, keepdims=True)
        rec = iota_s == s
        gx = gx + jnp.where(rec, px - nxs, 0.0)
        gy = gy + jnp.where(rec, py - nys, 0.0)
        gz = gz + jnp.where(rec, pz - nzs, 0.0)
        return gx, gy, gz

    zks = jnp.zeros((K, S), dtype=jnp.float32)
    gx, gy, gz = jax.lax.fori_loop(0, S, body, (zks, zks, zks))
    gx_ref[...] = gx[None]
    gy_ref[...] = gy[None]
    gz_ref[...] = gz[None]


def _run_select(xyz, nx, ny, nz):
    out = jax.ShapeDtypeStruct((B, K, S), jnp.float32)
    nx3 = nx[:, None, :]
    ny3 = ny[:, None, :]
    nz3 = nz[:, None, :]
    return pl.pallas_call(
        _select_kernel,
        grid=(B,),
        in_specs=[
            pl.BlockSpec((1, 3, N), lambda b: (b, 0, 0)),
            pl.BlockSpec((1, 1, S), lambda b: (b, 0, 0)),
            pl.BlockSpec((1, 1, S), lambda b: (b, 0, 0)),
            pl.BlockSpec((1, 1, S), lambda b: (b, 0, 0)),
        ],
        out_specs=(
            pl.BlockSpec((1, K, S), lambda b: (b, 0, 0)),
            pl.BlockSpec((1, K, S), lambda b: (b, 0, 0)),
            pl.BlockSpec((1, K, S), lambda b: (b, 0, 0)),
        ),
        out_shape=(out, out, out),
        scratch_shapes=[pltpu.VMEM((K, N), jnp.float32)],
    )(xyz, nx3, ny3, nz3)


M = K * S


def _mlp_kernel(g_ref, w1_ref, b1_ref, g1_ref, be1_ref,
                w2_ref, b2_ref, g2_ref, be2_ref, out_ref, h1_ref, y2_ref):
    w1 = w1_ref[...]
    w2 = w2_ref[...]
    nelem = jnp.float32(B * S * K)
    dot = functools.partial(jnp.dot, preferred_element_type=jnp.float32)

    s1 = jnp.zeros((C1, 1), jnp.float32)
    for b in range(B):
        y1 = dot(w1, g_ref[b]) + b1_ref[...]
        h1_ref[pl.ds(b * C1, C1), :] = y1
        s1 = s1 + jnp.sum(y1, axis=1, keepdims=True)
    mean1 = s1 / nelem
    v1 = jnp.zeros((C1, 1), jnp.float32)
    for b in range(B):
        dev = h1_ref[pl.ds(b * C1, C1), :] - mean1
        v1 = v1 + jnp.sum(dev * dev, axis=1, keepdims=True)
    inv1 = g1_ref[...] / jnp.sqrt(v1 / nelem + 1e-5)

    s2 = jnp.zeros((C2, 1), jnp.float32)
    for b in range(B):
        h1 = jnp.maximum((h1_ref[pl.ds(b * C1, C1), :] - mean1) * inv1
                         + be1_ref[...], 0.0)
        y2 = dot(w2, h1) + b2_ref[...]
        y2_ref[pl.ds(b * C2, C2), :] = y2
        s2 = s2 + jnp.sum(y2, axis=1, keepdims=True)
    mean2 = s2 / nelem
    v2 = jnp.zeros((C2, 1), jnp.float32)
    for b in range(B):
        dev = y2_ref[pl.ds(b * C2, C2), :] - mean2
        v2 = v2 + jnp.sum(dev * dev, axis=1, keepdims=True)
    inv2 = g2_ref[...] / jnp.sqrt(v2 / nelem + 1e-5)

    for b in range(B):
        h2 = jnp.maximum((y2_ref[pl.ds(b * C2, C2), :] - mean2) * inv2
                         + be2_ref[...], 0.0)
        acc = h2[:, 0:S]
        for k in range(1, K):
            acc = jnp.maximum(acc, h2[:, k * S:(k + 1) * S])
        out_ref[b] = acc


def _run_mlp(g, W1, b1, g1, be1, W2, b2, g2, be2):
    col = lambda v: v[:, None]
    return pl.pallas_call(
        _mlp_kernel,
        out_shape=jax.ShapeDtypeStruct((B, C2, S), jnp.float32),
        scratch_shapes=[
            pltpu.VMEM((B * C1, M), jnp.float32),
            pltpu.VMEM((B * C2, M), jnp.float32),
        ],
    )(g, W1, col(b1), col(g1), col(be1), W2, col(b2), col(g2), col(be2))


@jax.jit
def kernel(xyz, W1, b1, g1, be1, W2, b2, g2, be2):
    init_far = jax.random.randint(jax.random.key(42), (B,), 0, N,
                                  dtype=jnp.int32)[:, None]
    nx, ny, nz = _run_fps(xyz, init_far)
    gx, gy, gz = _run_select(xyz, nx, ny, nz)
    g = jnp.concatenate([gx.reshape(B, 1, M), gy.reshape(B, 1, M),
                         gz.reshape(B, 1, M)], axis=1)
    features = _run_mlp(g, W1, b1, g1, be1, W2, b2, g2, be2)
    new_xyz = jnp.concatenate([nx[:, None, :], ny[:, None, :], nz[:, None, :]],
                              axis=1)
    return features, new_xyz


# speedup vs baseline: 8.9166x; 1.7883x over previous
import functools

import jax
import jax.numpy as jnp
from jax.experimental import pallas as pl
from jax.experimental.pallas import tpu as pltpu
from jax.experimental.pallas import tpu_sc as plsc

B = 8
N = 16384
S = 256
K = 32
C1 = 32
C2 = 64
BIG = 3e38


def _fps_kernel(xyz_ref, init_ref, nx_ref, ny_ref, nz_ref):
    x = xyz_ref[:, 0, :]
    y = xyz_ref[:, 1, :]
    z = xyz_ref[:, 2, :]
    iota_n = jax.lax.broadcasted_iota(jnp.int32, (B, N), 1)
    iota_s = jax.lax.broadcasted_iota(jnp.int32, (B, S), 1)

    def body(i, carry):
        distance, far, ax, ay, az = carry
        sel = iota_n == far
        cx = jnp.sum(jnp.where(sel, x, 0.0), axis=1, keepdims=True)
        cy = jnp.sum(jnp.where(sel, y, 0.0), axis=1, keepdims=True)
        cz = jnp.sum(jnp.where(sel, z, 0.0), axis=1, keepdims=True)
        rec = iota_s == i
        ax = ax + jnp.where(rec, cx, 0.0)
        ay = ay + jnp.where(rec, cy, 0.0)
        az = az + jnp.where(rec, cz, 0.0)
        dx = x - cx
        dy = y - cy
        dz = z - cz
        dist = (dx * dx + dy * dy) + dz * dz
        distance = jnp.where(dist < distance, dist, distance)
        m = jnp.max(distance, axis=1, keepdims=True)
        far = jnp.min(jnp.where(distance == m, iota_n, N), axis=1, keepdims=True)
        return distance, far, ax, ay, az

    dist0 = jnp.full((B, N), 1e10, dtype=jnp.float32)
    zs = jnp.zeros((B, S), dtype=jnp.float32)
    far0 = init_ref[...]
    _, _, ax, ay, az = jax.lax.fori_loop(0, S, body, (dist0, far0, zs, zs, zs))
    nx_ref[...] = ax
    ny_ref[...] = ay
    nz_ref[...] = az


def _run_fps(xyz, init_far):
    out = jax.ShapeDtypeStruct((B, S), jnp.float32)
    return pl.pallas_call(
        _fps_kernel,
        out_shape=(out, out, out),
    )(xyz, init_far)


def _select_kernel(xyz_ref, nx_ref, ny_ref, nz_ref, gi_ref, d_ref):
    x = xyz_ref[:, 0, :]
    y = xyz_ref[:, 1, :]
    z = xyz_ref[:, 2, :]
    nxr = nx_ref[:, 0, :]
    nyr = ny_ref[:, 0, :]
    nzr = nz_ref[:, 0, :]

    io_l = jax.lax.broadcasted_iota(jnp.int32, (K, K), 1)
    io_s = jax.lax.broadcasted_iota(jnp.int32, (K, K), 0)
    t = io_l == io_s
    cx = jnp.sum(jnp.where(t, nxr[:, :K], 0.0), axis=1, keepdims=True)
    cy = jnp.sum(jnp.where(t, nyr[:, :K], 0.0), axis=1, keepdims=True)
    cz = jnp.sum(jnp.where(t, nzr[:, :K], 0.0), axis=1, keepdims=True)

    dx = x - cx
    dy = y - cy
    dz = z - cz
    d_ref[...] = (dx * dx + dy * dy) + dz * dz

    iota_n = jax.lax.broadcasted_iota(jnp.int32, (K, N), 1)
    iota_s = jax.lax.broadcasted_iota(jnp.int32, (K, S), 1)

    def body(s, gi):
        d = d_ref[...]
        m = jnp.min(d, axis=1, keepdims=True)
        idx = jnp.min(jnp.where(d == m, iota_n, N), axis=1, keepdims=True)
        d_ref[...] = jnp.where(iota_n == idx, BIG, d)
        return gi + jnp.where(iota_s == s, idx, 0)

    gi = jax.lax.fori_loop(0, S, body, jnp.zeros((K, S), dtype=jnp.int32))
    gi_ref[...] = gi[None]


def _run_select(xyz, nx, ny, nz):
    nx3 = nx[:, None, :]
    ny3 = ny[:, None, :]
    nz3 = nz[:, None, :]
    return pl.pallas_call(
        _select_kernel,
        grid=(B,),
        in_specs=[
            pl.BlockSpec((1, 3, N), lambda b: (b, 0, 0)),
            pl.BlockSpec((1, 1, S), lambda b: (b, 0, 0)),
            pl.BlockSpec((1, 1, S), lambda b: (b, 0, 0)),
            pl.BlockSpec((1, 1, S), lambda b: (b, 0, 0)),
        ],
        out_specs=pl.BlockSpec((1, K, S), lambda b: (b, 0, 0)),
        out_shape=jax.ShapeDtypeStruct((B, K, S), jnp.int32),
        scratch_shapes=[pltpu.VMEM((K, N), jnp.float32)],
    )(xyz, nx3, ny3, nz3)


M = K * S
PW = 4
CHUNK = M // PW
L = 16


def _sc_gather_kernel(xyz_hbm, idx_hbm, out_hbm, xr_v, yr_v, zr_v, idx_v,
                      ox_v, oy_v, oz_v):
    wid = jax.lax.axis_index("s") * 2 + jax.lax.axis_index("c")
    b = wid // PW
    part = wid % PW
    pltpu.sync_copy(xyz_hbm.at[pl.ds((b * 3 + 0) * N, N)], xr_v)
    pltpu.sync_copy(xyz_hbm.at[pl.ds((b * 3 + 1) * N, N)], yr_v)
    pltpu.sync_copy(xyz_hbm.at[pl.ds((b * 3 + 2) * N, N)], zr_v)
    off = part * CHUNK
    pltpu.sync_copy(idx_hbm.at[pl.ds(b * M + off, CHUNK)], idx_v)

    def body(i, carry):
        iv = idx_v[pl.ds(i * L, L)]
        ox_v[pl.ds(i * L, L)] = plsc.load_gather(xr_v, [iv])
        oy_v[pl.ds(i * L, L)] = plsc.load_gather(yr_v, [iv])
        oz_v[pl.ds(i * L, L)] = plsc.load_gather(zr_v, [iv])
        return carry

    jax.lax.fori_loop(0, CHUNK // L, body, 0)
    pltpu.sync_copy(ox_v, out_hbm.at[pl.ds((b * 3 + 0) * M + off, CHUNK)])
    pltpu.sync_copy(oy_v, out_hbm.at[pl.ds((b * 3 + 1) * M + off, CHUNK)])
    pltpu.sync_copy(oz_v, out_hbm.at[pl.ds((b * 3 + 2) * M + off, CHUNK)])


def _run_sc_gather(xyz, gidx):
    mesh = plsc.VectorSubcoreMesh(core_axis_name="c", subcore_axis_name="s")
    f = functools.partial(
        pl.kernel,
        mesh=mesh,
        compiler_params=pltpu.CompilerParams(needs_layout_passes=False),
        out_type=jax.ShapeDtypeStruct((B * 3 * M,), jnp.float32),
        scratch_types=[
            pltpu.VMEM((N,), jnp.float32),
            pltpu.VMEM((N,), jnp.float32),
            pltpu.VMEM((N,), jnp.float32),
            pltpu.VMEM((CHUNK,), jnp.int32),
            pltpu.VMEM((CHUNK,), jnp.float32),
            pltpu.VMEM((CHUNK,), jnp.float32),
            pltpu.VMEM((CHUNK,), jnp.float32),
        ],
    )(_sc_gather_kernel)
    return f(xyz.reshape(B * 3 * N), gidx.reshape(B * M)).reshape(B, 3, M)


def _mlp_kernel(g_ref, nx_ref, ny_ref, nz_ref, w1_ref, b1_ref, g1_ref,
                be1_ref, w2_ref, b2_ref, g2_ref, be2_ref, out_ref, h1_ref,
                y2_ref):
    w1 = w1_ref[...]
    w2 = w2_ref[...]
    nelem = jnp.float32(B * S * K)
    dot = functools.partial(jnp.dot, preferred_element_type=jnp.float32)

    s1 = jnp.zeros((C1, 1), jnp.float32)
    for b in range(B):
        c3 = jnp.concatenate([nx_ref[b], ny_ref[b], nz_ref[b]], axis=0)
        ctile = jnp.concatenate([c3] * K, axis=1)
        y1 = dot(w1, g_ref[b] - ctile) + b1_ref[...]
        h1_ref[pl.ds(b * C1, C1), :] = y1
        s1 = s1 + jnp.sum(y1, axis=1, keepdims=True)
    mean1 = s1 / nelem
    v1 = jnp.zeros((C1, 1), jnp.float32)
    for b in range(B):
        dev = h1_ref[pl.ds(b * C1, C1), :] - mean1
        v1 = v1 + jnp.sum(dev * dev, axis=1, keepdims=True)
    inv1 = g1_ref[...] / jnp.sqrt(v1 / nelem + 1e-5)

    s2 = jnp.zeros((C2, 1), jnp.float32)
    for b in range(B):
        h1 = jnp.maximum((h1_ref[pl.ds(b * C1, C1), :] - mean1) * inv1
                         + be1_ref[...], 0.0)
        y2 = dot(w2, h1) + b2_ref[...]
        y2_ref[pl.ds(b * C2, C2), :] = y2
        s2 = s2 + jnp.sum(y2, axis=1, keepdims=True)
    mean2 = s2 / nelem
    v2 = jnp.zeros((C2, 1), jnp.float32)
    for b in range(B):
        dev = y2_ref[pl.ds(b * C2, C2), :] - mean2
        v2 = v2 + jnp.sum(dev * dev, axis=1, keepdims=True)
    inv2 = g2_ref[...] / jnp.sqrt(v2 / nelem + 1e-5)

    for b in range(B):
        h2 = jnp.maximum((y2_ref[pl.ds(b * C2, C2), :] - mean2) * inv2
                         + be2_ref[...], 0.0)
        acc = h2[:, 0:S]
        for k in range(1, K):
            acc = jnp.maximum(acc, h2[:, k * S:(k + 1) * S])
        out_ref[b] = acc


def _run_mlp(g, nx, ny, nz, W1, b1, g1, be1, W2, b2, g2, be2):
    col = lambda v: v[:, None]
    return pl.pallas_call(
        _mlp_kernel,
        out_shape=jax.ShapeDtypeStruct((B, C2, S), jnp.float32),
        scratch_shapes=[
            pltpu.VMEM((B * C1, M), jnp.float32),
            pltpu.VMEM((B * C2, M), jnp.float32),
        ],
    )(g, nx[:, None, :], ny[:, None, :], nz[:, None, :], W1, col(b1),
      col(g1), col(be1), W2, col(b2), col(g2), col(be2))


@jax.jit
def kernel(xyz, W1, b1, g1, be1, W2, b2, g2, be2):
    init_far = jax.random.randint(jax.random.key(42), (B,), 0, N,
                                  dtype=jnp.int32)[:, None]
    nx, ny, nz = _run_fps(xyz, init_far)
    gidx = _run_select(xyz, nx, ny, nz)
    g = _run_sc_gather(xyz, gidx)
    features = _run_mlp(g, nx, ny, nz, W1, b1, g1, be1, W2, b2, g2, be2)
    new_xyz = jnp.concatenate([nx[:, None, :], ny[:, None, :], nz[:, None, :]],
                              axis=1)
    return features, new_xyz
